# baseline (speedup 1.0000x reference)
import jax
import jax.numpy as jnp
from jax.experimental import pallas as pl
from jax.experimental.pallas import tpu as pltpu

_BM = 400
_NBUF = 3


def _mm_kernel(mask_hbm, oh_ref, out_ref, b0, b1, b2, sems):
    i = pl.program_id(0)
    n_blocks = pl.num_programs(0)
    bufs = (b0, b1, b2)

    def start(block, buf, sem):
        pltpu.make_async_copy(
            mask_hbm.at[pl.ds(block * _BM, _BM), :], buf, sem).start()

    @pl.when(i == 0)
    def _():
        for j in range(_NBUF):
            start(j, bufs[j], sems.at[j])

    oh = oh_ref[...].astype(jnp.bfloat16)
    for j in range(_NBUF):
        @pl.when(i % _NBUF == j)
        def _(j=j):
            pltpu.make_async_copy(
                mask_hbm.at[pl.ds(i * _BM, _BM), :], bufs[j], sems.at[j]).wait()
            out_ref[pl.ds(i * _BM, _BM), :] = jnp.dot(
                bufs[j][...].astype(jnp.bfloat16), oh,
                preferred_element_type=jnp.float32)

            @pl.when(i + _NBUF < n_blocks)
            def _():
                start(i + _NBUF, bufs[j], sems.at[j])


def kernel(mask_matrix, x, one_hot_h):
    del x
    n_rows, k = mask_matrix.shape
    n_types = one_hot_h.shape[1]
    return pl.pallas_call(
        _mm_kernel,
        grid=(n_rows // _BM,),
        in_specs=[
            pl.BlockSpec(memory_space=pl.ANY),
            pl.BlockSpec((k, n_types), lambda i: (0, 0)),
        ],
        out_specs=pl.BlockSpec((n_rows, n_types), lambda i: (0, 0)),
        out_shape=jax.ShapeDtypeStruct((n_rows, n_types), jnp.float32),
        scratch_shapes=[
            pltpu.VMEM((_BM, k), jnp.float32),
            pltpu.VMEM((_BM, k), jnp.float32),
            pltpu.VMEM((_BM, k), jnp.float32),
            pltpu.SemaphoreType.DMA((_NBUF,)),
        ],
        compiler_params=pltpu.CompilerParams(
            dimension_semantics=("arbitrary",),
        ),
    )(mask_matrix, one_hot_h)
